# Initial kernel scaffold; baseline (speedup 1.0000x reference)
import jax
import jax.numpy as jnp
from jax.experimental import pallas as pl


def kernel(features, atten, Wg1, bg1, Wg2, bg2, Wfc, bfc, W1, b1, gamma1, beta1, W2, b2):
    raise NotImplementedError("write your pallas kernel here")


# speedup vs baseline: 1.0713x; 1.0713x over previous
import functools

import jax
import jax.numpy as jnp
from jax.experimental import pallas as pl


def _round_to_f16(x):
    u = jax.lax.bitcast_convert_type(x, jnp.int32)
    add = jnp.int32(0x0FFF) + ((u >> 13) & jnp.int32(1))
    u = (u + add) & jnp.int32(~0x1FFF)
    return jax.lax.bitcast_convert_type(u, jnp.float32)


def _stage_a(f_ref, wg1_ref, bg1_ref, wg2_ref, bg2_ref, w1_ref, b1_ref,
             wfc_ref, bfc_ref, x1_ref, feats_ref, stats_ref, *, n, kk, kp):
    f = f_ref[0]
    dn = (((1,), (1,)), ((), ()))

    h = jax.lax.dot_general(f, wg1_ref[...], dn,
                            preferred_element_type=jnp.float32)
    h = jnp.maximum(h + bg1_ref[...], 0.0)
    w_all = jax.lax.dot_general(h, wg2_ref[...], dn,
                                preferred_element_type=jnp.float32)
    w_col = w_all[:, :1] + bg2_ref[0, 0]

    ii = jax.lax.broadcasted_iota(jnp.int32, (n, n), 0)
    jj = jax.lax.broadcasted_iota(jnp.int32, (n, n), 1)

    i_col = jax.lax.broadcasted_iota(jnp.int32, (n, 1), 0)
    w_col = jnp.where(i_col == 0, -jnp.finfo(jnp.float32).max, w_col)

    wq = w_col + 0.0
    u = jax.lax.bitcast_convert_type(wq, jnp.int32)
    key_col = jnp.where(u >= 0, u, u ^ jnp.int32(0x7FFFFFFF))

    eye = (ii == jj).astype(jnp.float32)

    def _xpose_small(c):
        r = jax.lax.dot_general(c.astype(jnp.float32), eye,
                                (((0,), (0,)), ((), ())),
                                preferred_element_type=jnp.float32)
        return r.astype(jnp.int32)

    m255 = jnp.int32(255)
    key_row = ((_xpose_small(key_col >> 24) << 24)
               | (_xpose_small((key_col >> 16) & m255) << 16)
               | (_xpose_small((key_col >> 8) & m255) << 8)
               | _xpose_small(key_col & m255))

    beats = (key_col > key_row) | ((key_col == key_row) & (ii < jj))
    rank = jnp.sum(beats.astype(jnp.float32), axis=0, keepdims=True)
    selected = rank < float(kk)

    lt = (ii <= jj).astype(jnp.float32)
    pos = jax.lax.dot_general(selected.astype(jnp.float32), lt,
                              (((1,), (0,)), ((), ())),
                              preferred_element_type=jnp.float32) - 1.0

    r_iota = jax.lax.broadcasted_iota(jnp.int32, (kp, 1), 0).astype(jnp.float32)
    m = ((pos == r_iota) & selected).astype(jnp.float32)

    sel = jax.lax.dot_general(m, f, (((1,), (0,)), ((), ())),
                              preferred_element_type=jnp.float32)
    ssq = jnp.sum(f * f, axis=1, keepdims=True)
    ssq_sel = jax.lax.dot_general(m, ssq, (((1,), (0,)), ((), ())),
                                  preferred_element_type=jnp.float32)
    sel_n = sel / (jnp.sqrt(ssq_sel) + 1e-8)

    x1 = jax.lax.dot_general(sel_n, w1_ref[...], dn,
                             preferred_element_type=jnp.float32)
    x1 = x1 + b1_ref[...]
    valid = jax.lax.broadcasted_iota(jnp.int32, (kp, 1), 0) < kk
    x1v = jnp.where(valid, x1, 0.0)
    s1 = jnp.sum(x1v, axis=0, keepdims=True)
    s2 = jnp.sum(x1v * x1v, axis=0, keepdims=True)
    stats_ref[0] = jnp.concatenate([s1, s2], axis=0)
    x1_ref[0] = x1

    sel16 = _round_to_f16(sel_n)
    wfc16 = _round_to_f16(wfc_ref[...])
    fr = jax.lax.dot_general(sel16, wfc16, dn,
                             preferred_element_type=jnp.float32)
    fr16 = _round_to_f16(_round_to_f16(fr) + _round_to_f16(bfc_ref[...]))
    feats_ref[0] = fr16


def _stage_b(x1_ref, feats_ref, stats_ref, w2_ref, b2_ref, g1_ref, be1_ref,
             out_ref, *, kk, r_total):
    stats = stats_ref[...]
    s1 = jnp.sum(stats[:, 0, :], axis=0, keepdims=True)
    s2 = jnp.sum(stats[:, 1, :], axis=0, keepdims=True)
    mu = s1 / r_total
    var = s2 / r_total - mu * mu

    x1 = x1_ref[0]
    xn = (x1 - mu) / jnp.sqrt(var + 1e-5) * g1_ref[...] + be1_ref[...]
    xn = jnp.maximum(xn, 0.0)
    y = jax.lax.dot_general(xn, w2_ref[...], (((1,), (1,)), ((), ())),
                            preferred_element_type=jnp.float32)
    y = y + b2_ref[...]
    out = y + feats_ref[0]
    out_ref[0] = out[:kk]


def kernel(features, atten, Wg1, bg1, Wg2, bg2, Wfc, bfc, W1, b1, gamma1,
           beta1, W2, b2):
    B, N, D = features.shape
    E = Wfc.shape[0]
    H = W1.shape[0]
    KK = int((atten.shape[1] - 1) * 0.3)
    KP = ((KK + 7) // 8) * 8
    R = B * KK

    Wg2p = jnp.zeros((128, D), jnp.float32).at[0].set(Wg2[0])
    bg1r = bg1.reshape(1, D)
    bg2r = bg2.reshape(1, 1)
    b1r = b1.reshape(1, H)
    bfcr = bfc.reshape(1, E)
    b2r = b2.reshape(1, E)
    g1r = gamma1.reshape(1, H)
    be1r = beta1.reshape(1, H)

    const = lambda *dims: pl.BlockSpec(dims, lambda b: (0,) * len(dims))
    perb = lambda *dims: pl.BlockSpec((1,) + dims,
                                      lambda b: (b,) + (0,) * len(dims))

    x1, feats, stats = pl.pallas_call(
        functools.partial(_stage_a, n=N, kk=KK, kp=KP),
        grid=(B,),
        in_specs=[
            perb(N, D),
            const(D, D),
            const(1, D),
            const(128, D),
            const(1, 1),
            const(H, D),
            const(1, H),
            const(E, D),
            const(1, E),
        ],
        out_specs=[perb(KP, H), perb(KP, E), perb(2, H)],
        out_shape=[
            jax.ShapeDtypeStruct((B, KP, H), jnp.float32),
            jax.ShapeDtypeStruct((B, KP, E), jnp.float32),
            jax.ShapeDtypeStruct((B, 2, H), jnp.float32),
        ],
    )(features, Wg1, bg1r, Wg2p, bg2r, W1, b1r, Wfc, bfcr)

    out = pl.pallas_call(
        functools.partial(_stage_b, kk=KK, r_total=float(R)),
        grid=(B,),
        in_specs=[
            perb(KP, H),
            perb(KP, E),
            const(B, 2, H),
            const(E, H),
            const(1, E),
            const(1, H),
            const(1, H),
        ],
        out_specs=perb(KK, E),
        out_shape=jax.ShapeDtypeStruct((B, KK, E), jnp.float32),
    )(x1, feats, stats, W2, b2r, g1r, be1r)

    return out


# speedup vs baseline: 1.1205x; 1.0459x over previous
import functools

import jax
import jax.numpy as jnp
from jax.experimental import pallas as pl


def _round_to_f16(x):
    u = jax.lax.bitcast_convert_type(x, jnp.int32)
    add = jnp.int32(0x0FFF) + ((u >> 13) & jnp.int32(1))
    u = (u + add) & jnp.int32(~0x1FFF)
    return jax.lax.bitcast_convert_type(u, jnp.float32)


def _stage_a(f_ref, wg1_ref, bg1_ref, wg2_ref, bg2_ref, w1_ref, b1_ref,
             seln_ref, x1_ref, stats_ref, *, n, kk, kp):
    f = f_ref[0]
    dn = (((1,), (1,)), ((), ()))

    h = jax.lax.dot_general(f, wg1_ref[...], dn,
                            preferred_element_type=jnp.float32)
    h = jnp.maximum(h + bg1_ref[...], 0.0)
    w_all = jax.lax.dot_general(h, wg2_ref[...], dn,
                                preferred_element_type=jnp.float32)
    w_col = w_all[:, :1] + bg2_ref[0, 0]

    ii = jax.lax.broadcasted_iota(jnp.int32, (n, n), 0)
    jj = jax.lax.broadcasted_iota(jnp.int32, (n, n), 1)

    i_col = jax.lax.broadcasted_iota(jnp.int32, (n, 1), 0)
    w_col = jnp.where(i_col == 0, -jnp.finfo(jnp.float32).max, w_col)

    wq = w_col + 0.0
    u = jax.lax.bitcast_convert_type(wq, jnp.int32)
    key_col = jnp.where(u >= 0, u, u ^ jnp.int32(0x7FFFFFFF))

    eye = (ii == jj).astype(jnp.float32)

    def _xpose_small(c):
        r = jax.lax.dot_general(c.astype(jnp.float32), eye,
                                (((0,), (0,)), ((), ())),
                                preferred_element_type=jnp.float32)
        return r.astype(jnp.int32)

    m255 = jnp.int32(255)
    key_row = ((_xpose_small(key_col >> 24) << 24)
               | (_xpose_small((key_col >> 16) & m255) << 16)
               | (_xpose_small((key_col >> 8) & m255) << 8)
               | _xpose_small(key_col & m255))

    beats = (key_col > key_row) | ((key_col == key_row) & (ii < jj))
    rank = jnp.sum(beats.astype(jnp.float32), axis=0, keepdims=True)
    selected = rank < float(kk)

    lt = (ii <= jj).astype(jnp.float32)
    pos = jax.lax.dot_general(selected.astype(jnp.float32), lt,
                              (((1,), (0,)), ((), ())),
                              preferred_element_type=jnp.float32) - 1.0

    r_iota = jax.lax.broadcasted_iota(jnp.int32, (kp, 1), 0).astype(jnp.float32)
    m = ((pos == r_iota) & selected).astype(jnp.float32)

    sel = jax.lax.dot_general(m, f, (((1,), (0,)), ((), ())),
                              preferred_element_type=jnp.float32)
    ssq = jnp.sum(f * f, axis=1, keepdims=True)
    ssq_sel = jax.lax.dot_general(m, ssq, (((1,), (0,)), ((), ())),
                                  preferred_element_type=jnp.float32)
    sel_n = sel / (jnp.sqrt(ssq_sel) + 1e-8)
    seln_ref[0] = sel_n

    x1 = jax.lax.dot_general(sel_n, w1_ref[...], dn,
                             preferred_element_type=jnp.float32)
    x1 = x1 + b1_ref[...]
    valid = jax.lax.broadcasted_iota(jnp.int32, (kp, 1), 0) < kk
    x1v = jnp.where(valid, x1, 0.0)
    s1 = jnp.sum(x1v, axis=0, keepdims=True)
    s2 = jnp.sum(x1v * x1v, axis=0, keepdims=True)
    stats_ref[0] = jnp.concatenate([s1, s2], axis=0)
    x1_ref[0] = x1


def _stage_b(seln_ref, x1_ref, stats_ref, wfc_ref, bfc_ref, w2_ref, b2_ref,
             g1_ref, be1_ref, out_ref, *, kk, r_total):
    dn = (((1,), (1,)), ((), ()))
    stats = stats_ref[...]
    s1 = jnp.sum(stats[:, 0, :], axis=0, keepdims=True)
    s2 = jnp.sum(stats[:, 1, :], axis=0, keepdims=True)
    mu = s1 / r_total
    var = s2 / r_total - mu * mu

    x1 = x1_ref[0]
    xn = (x1 - mu) / jnp.sqrt(var + 1e-5) * g1_ref[...] + be1_ref[...]
    xn = jnp.maximum(xn, 0.0)
    y = jax.lax.dot_general(xn, w2_ref[...], dn,
                            preferred_element_type=jnp.float32)
    y = y + b2_ref[...]

    sel16 = _round_to_f16(seln_ref[0])
    wfc16 = _round_to_f16(wfc_ref[...])
    fr = jax.lax.dot_general(sel16, wfc16, dn,
                             preferred_element_type=jnp.float32)
    fr16 = _round_to_f16(_round_to_f16(fr) + _round_to_f16(bfc_ref[...]))

    out = y + fr16
    out_ref[0] = out[:kk]


def kernel(features, atten, Wg1, bg1, Wg2, bg2, Wfc, bfc, W1, b1, gamma1,
           beta1, W2, b2):
    B, N, D = features.shape
    E = Wfc.shape[0]
    H = W1.shape[0]
    KK = int((atten.shape[1] - 1) * 0.3)
    KP = ((KK + 7) // 8) * 8
    R = B * KK

    Wg2p = jnp.zeros((128, D), jnp.float32).at[0].set(Wg2[0])
    bg1r = bg1.reshape(1, D)
    bg2r = bg2.reshape(1, 1)
    b1r = b1.reshape(1, H)
    bfcr = bfc.reshape(1, E)
    b2r = b2.reshape(1, E)
    g1r = gamma1.reshape(1, H)
    be1r = beta1.reshape(1, H)

    const = lambda *dims: pl.BlockSpec(dims, lambda b: (0,) * len(dims))
    perb = lambda *dims: pl.BlockSpec((1,) + dims,
                                      lambda b: (b,) + (0,) * len(dims))

    seln, x1, stats = pl.pallas_call(
        functools.partial(_stage_a, n=N, kk=KK, kp=KP),
        grid=(B,),
        in_specs=[
            perb(N, D),
            const(D, D),
            const(1, D),
            const(128, D),
            const(1, 1),
            const(H, D),
            const(1, H),
        ],
        out_specs=[perb(KP, D), perb(KP, H), perb(2, H)],
        out_shape=[
            jax.ShapeDtypeStruct((B, KP, D), jnp.float32),
            jax.ShapeDtypeStruct((B, KP, H), jnp.float32),
            jax.ShapeDtypeStruct((B, 2, H), jnp.float32),
        ],
    )(features, Wg1, bg1r, Wg2p, bg2r, W1, b1r)

    out = pl.pallas_call(
        functools.partial(_stage_b, kk=KK, r_total=float(R)),
        grid=(B,),
        in_specs=[
            perb(KP, D),
            perb(KP, H),
            const(B, 2, H),
            const(E, D),
            const(1, E),
            const(E, H),
            const(1, E),
            const(1, H),
            const(1, H),
        ],
        out_specs=perb(KK, E),
        out_shape=jax.ShapeDtypeStruct((B, KK, E), jnp.float32),
    )(seln, x1, stats, Wfc, bfcr, W2, b2r, g1r, be1r)

    return out
